# baseline (speedup 1.0000x reference)
import functools

import jax
import jax.numpy as jnp
from jax import lax
from jax.experimental import pallas as pl
from jax.experimental.pallas import tpu as pltpu
from jax.experimental.pallas import tpu_sc as plsc

_N = 65536
_K = 8192
_H = 4096
_D = 32
_B = 512
_NBLK = _N // _B

_NC = 2
_NS = 16
_NW = _NC * _NS
_BW = _N // _NW
_CH = 128
_NCH = _BW // _CH


def _vq_body(x_ref, x2_ref, wt_ref, w2_ref,
             idx_ref, loss_ref, perp_ref,
             cnt_acc, loss_acc):
    i = pl.program_id(0)

    @pl.when(i == 0)
    def _init():
        cnt_acc[...] = jnp.zeros_like(cnt_acc)
        loss_acc[...] = jnp.zeros_like(loss_acc)

    x = x_ref[...]
    xb = x.astype(jnp.bfloat16)
    s2 = jnp.dot(xb, wt_ref[...], preferred_element_type=jnp.float32)
    x2 = x2_ref[...]
    w2 = w2_ref[...]
    dist = (x2 + w2) - s2

    d0 = dist[:, :_H]
    d1 = dist[:, _H:]
    m0 = jnp.min(d0, axis=1, keepdims=True)
    m1 = jnp.min(d1, axis=1, keepdims=True)
    colh = jax.lax.broadcasted_iota(jnp.int32, (_B, _H), 1)
    i0 = jnp.min(jnp.where(d0 == m0, colh, _K), axis=1)
    i1 = jnp.min(jnp.where(d1 == m1, colh + _H, _K), axis=1)

    m0b = m0.astype(jnp.bfloat16).astype(jnp.float32)
    take1 = m1 < m0b
    idx = jnp.where(take1[:, 0], i1, i0)
    idx_ref[0, 0, :] = idx

    ld = jnp.where(take1, m1, m0)

    col = jax.lax.broadcasted_iota(jnp.int32, (_B, _K), 1)
    onehot = (col == idx[:, None]).astype(jnp.float32)
    cnt_acc[...] += jnp.sum(onehot, axis=0)[None, :]
    loss_acc[...] += jnp.sum(ld).reshape(1, 1)

    @pl.when(i == _NBLK - 1)
    def _fini():
        loss_ref[...] = loss_acc[...] * (1.25 / (_N * _D))
        avg = cnt_acc[...] * (1.0 / _N)
        ent = avg * jnp.log(avg + 1e-10)
        perp_ref[...] = jnp.exp(-jnp.sum(ent)).reshape(1, 1)


def _sc_gather_body(w_hbm, idx_hbm, out_hbm, idx_v, rows_v, sem):
    wid = lax.axis_index("s") * _NC + lax.axis_index("c")
    base = wid * _BW
    for c in range(_NCH):
        off = base + c * _CH
        pltpu.sync_copy(idx_hbm.at[pl.ds(off, _CH)], idx_v)
        pltpu.async_copy(w_hbm.at[idx_v], rows_v, sem).wait()
        pltpu.sync_copy(rows_v, out_hbm.at[pl.ds(off, _CH)])


@jax.jit
def kernel(inputs, weight):
    input_shape = inputs.shape
    flat = inputs.reshape(-1, _D)
    x2 = jnp.sum(flat ** 2, axis=1, keepdims=True)
    w2 = jnp.sum(weight ** 2, axis=1)[None, :]
    wt = 2.0 * weight.T

    idx3, loss11, perp11 = pl.pallas_call(
        _vq_body,
        grid=(_NBLK,),
        in_specs=[
            pl.BlockSpec((_B, _D), lambda i: (i, 0)),
            pl.BlockSpec((_B, 1), lambda i: (i, 0)),
            pl.BlockSpec((_D, _K), lambda i: (0, 0)),
            pl.BlockSpec((1, _K), lambda i: (0, 0)),
        ],
        out_specs=[
            pl.BlockSpec((1, 1, _B), lambda i: (i, 0, 0)),
            pl.BlockSpec((1, 1), lambda i: (0, 0)),
            pl.BlockSpec((1, 1), lambda i: (0, 0)),
        ],
        out_shape=[
            jax.ShapeDtypeStruct((_NBLK, 1, _B), jnp.int32),
            jax.ShapeDtypeStruct((1, 1), jnp.float32),
            jax.ShapeDtypeStruct((1, 1), jnp.float32),
        ],
        scratch_shapes=[
            pltpu.VMEM((1, _K), jnp.float32),
            pltpu.VMEM((1, 1), jnp.float32),
        ],
        compiler_params=pltpu.CompilerParams(
            dimension_semantics=("arbitrary",),
        ),
    )(flat, x2, wt, w2)

    encoding_indices = idx3.reshape(_N)

    mesh = plsc.VectorSubcoreMesh(core_axis_name="c", subcore_axis_name="s")
    gather = functools.partial(
        pl.kernel,
        mesh=mesh,
        out_type=jax.ShapeDtypeStruct((_N, _D), jnp.float32),
        scratch_types=[
            pltpu.VMEM((_CH,), jnp.int32),
            pltpu.VMEM((_CH, _D), jnp.float32),
            pltpu.SemaphoreType.DMA,
        ],
        compiler_params=pltpu.CompilerParams(use_tc_tiling_on_sc=False),
    )(_sc_gather_body)
    q = gather(weight, encoding_indices)

    quantized_st = q.reshape(input_shape)
    loss = loss11.reshape(())
    perplexity = perp11.reshape(())
    return quantized_st, loss, perplexity, encoding_indices


# speedup vs baseline: 1.0550x; 1.0550x over previous
import functools

import jax
import jax.numpy as jnp
from jax import lax
from jax.experimental import pallas as pl
from jax.experimental.pallas import tpu as pltpu
from jax.experimental.pallas import tpu_sc as plsc

_N = 65536
_K = 8192
_H = 4096
_D = 32
_B = 512
_NBLK = _N // _B

_NC = 2
_NS = 16
_NW = _NC * _NS
_BW = _N // _NW
_CH = 128
_NCH = _BW // _CH


def _vq_body(x_ref, x2_ref, wt_ref, w2_ref,
             idx_ref, loss_ref, perp_ref,
             cnt_acc, loss_acc):
    i = pl.program_id(0)

    @pl.when(i == 0)
    def _init():
        cnt_acc[...] = jnp.zeros_like(cnt_acc)
        loss_acc[...] = jnp.zeros_like(loss_acc)

    x = x_ref[...]
    xb = x.astype(jnp.bfloat16)
    s = jnp.dot(xb, wt_ref[...], preferred_element_type=jnp.float32)
    x2 = x2_ref[...]
    w2 = w2_ref[...]
    dist = (x2 + w2) - 2.0 * s

    d0 = dist[:, :_H]
    d1 = dist[:, _H:]
    m0 = jnp.min(d0, axis=1, keepdims=True)
    m1 = jnp.min(d1, axis=1, keepdims=True)
    colh = jax.lax.broadcasted_iota(jnp.int32, (_B, _H), 1)
    i0 = jnp.min(jnp.where(d0 == m0, colh, _K), axis=1)
    i1 = jnp.min(jnp.where(d1 == m1, colh + _H, _K), axis=1)

    m0b = m0.astype(jnp.bfloat16).astype(jnp.float32)
    take1 = m1 < m0b
    idx = jnp.where(take1[:, 0], i1, i0)
    idx_ref[0, 0, :] = idx

    ld = jnp.where(take1, m1, m0)

    col = jax.lax.broadcasted_iota(jnp.int32, (_B, _K), 1)
    onehot = (col == idx[:, None]).astype(jnp.float32)
    cnt_acc[...] += jnp.sum(onehot, axis=0)[None, :]
    loss_acc[...] += jnp.sum(ld).reshape(1, 1)

    @pl.when(i == _NBLK - 1)
    def _fini():
        loss_ref[...] = loss_acc[...] * (1.25 / (_N * _D))
        avg = cnt_acc[...] * (1.0 / _N)
        ent = avg * jnp.log(avg + 1e-10)
        perp_ref[...] = jnp.exp(-jnp.sum(ent)).reshape(1, 1)


def _sc_gather_body(w_hbm, idx_hbm, out_hbm, idx_v, rows_v, sem):
    wid = lax.axis_index("s") * _NC + lax.axis_index("c")
    base = wid * _BW
    for c in range(_NCH):
        off = base + c * _CH
        pltpu.sync_copy(idx_hbm.at[pl.ds(off, _CH)], idx_v)
        pltpu.async_copy(w_hbm.at[idx_v], rows_v, sem).wait()
        pltpu.sync_copy(rows_v, out_hbm.at[pl.ds(off, _CH)])


@jax.jit
def kernel(inputs, weight):
    input_shape = inputs.shape
    flat = inputs.reshape(-1, _D)
    x2 = jnp.sum(flat ** 2, axis=1, keepdims=True)
    w2 = jnp.sum(weight ** 2, axis=1)[None, :]
    wt = weight.T

    idx3, loss11, perp11 = pl.pallas_call(
        _vq_body,
        grid=(_NBLK,),
        in_specs=[
            pl.BlockSpec((_B, _D), lambda i: (i, 0)),
            pl.BlockSpec((_B, 1), lambda i: (i, 0)),
            pl.BlockSpec((_D, _K), lambda i: (0, 0)),
            pl.BlockSpec((1, _K), lambda i: (0, 0)),
        ],
        out_specs=[
            pl.BlockSpec((1, 1, _B), lambda i: (i, 0, 0)),
            pl.BlockSpec((1, 1), lambda i: (0, 0)),
            pl.BlockSpec((1, 1), lambda i: (0, 0)),
        ],
        out_shape=[
            jax.ShapeDtypeStruct((_NBLK, 1, _B), jnp.int32),
            jax.ShapeDtypeStruct((1, 1), jnp.float32),
            jax.ShapeDtypeStruct((1, 1), jnp.float32),
        ],
        scratch_shapes=[
            pltpu.VMEM((1, _K), jnp.float32),
            pltpu.VMEM((1, 1), jnp.float32),
        ],
        compiler_params=pltpu.CompilerParams(
            dimension_semantics=("arbitrary",),
        ),
    )(flat, x2, wt, w2)

    encoding_indices = idx3.reshape(_N)

    mesh = plsc.VectorSubcoreMesh(core_axis_name="c", subcore_axis_name="s")
    gather = functools.partial(
        pl.kernel,
        mesh=mesh,
        out_type=jax.ShapeDtypeStruct((_N, _D), jnp.float32),
        scratch_types=[
            pltpu.VMEM((_CH,), jnp.int32),
            pltpu.VMEM((_CH, _D), jnp.float32),
            pltpu.SemaphoreType.DMA,
        ],
        compiler_params=pltpu.CompilerParams(use_tc_tiling_on_sc=False),
    )(_sc_gather_body)
    q = gather(weight, encoding_indices)

    quantized_st = q.reshape(input_shape)
    loss = loss11.reshape(())
    perplexity = perp11.reshape(())
    return quantized_st, loss, perplexity, encoding_indices
